# Initial kernel scaffold; baseline (speedup 1.0000x reference)
import jax
import jax.numpy as jnp
from jax.experimental import pallas as pl


def kernel(x, edge_index, W1, b1, W2, b2, Wc, bc):
    raise NotImplementedError("write your pallas kernel here")


# speedup vs baseline: 9.5663x; 9.5663x over previous
import functools

import jax
import jax.numpy as jnp
from jax import lax
from jax.experimental import pallas as pl
from jax.experimental.pallas import tpu as pltpu
from jax.experimental.pallas import tpu_sc as plsc

N = 10000
E = 160000
D = 256
H = 128
NCLS = 40

NC = 2
NS = 16

C = 128
EP_T = 10240
E_PAD = NS * EP_T
ITERS = EP_T // C
BPB = 8
NBODY = ITERS // (2 * BPB)
PADR = 240
ACC_R = N + PADR
ZPT = ACC_R // NS
ZR = 128
WPT = 624
WLAST = N - (NS - 1) * WPT


def _zero_vmem_rows(ref, nrows, width):
  def body(r, _):
    for k in range(width // 16):
      ref[r, pl.ds(k * 16, 16)] = jnp.zeros((16,), jnp.float32)
    return 0
  lax.fori_loop(0, nrows, body, 0)


def _edge_prime(srcp, dstp, tbl, idx, rows, sems, s):
  srcbA, srcbB, dstbA, dstbB = idx
  gs = sems[0:2]
  sA, dA, sB, dB = sems[2:6]
  pltpu.sync_copy(srcp.at[s, pl.ds(0, BPB)], srcbA)
  pltpu.sync_copy(dstp.at[s, pl.ds(0, BPB)], dstbA)
  pltpu.async_copy(tbl.at[srcbA.at[0]], rows.at[0], gs[0])
  pltpu.async_copy(srcp.at[s, pl.ds(BPB, BPB)], srcbB, sB)
  pltpu.async_copy(dstp.at[s, pl.ds(BPB, BPB)], dstbB, dB)


def _edge_pass(srcp, dstp, tbl, out, acc, idx, rows, sems, s,
               dacc=None, ones=None, degout=None):
  srcbA, srcbB, dstbA, dstbB = idx
  gs = sems[0:2]
  sA, dA, sB, dB = sems[2:6]

  def load_blk(bi, sbuf, dbuf, ssem, dsem):
    pltpu.async_copy(srcp.at[s, pl.ds(bi * BPB, BPB)], sbuf, ssem)
    pltpu.async_copy(dstp.at[s, pl.ds(bi * BPB, BPB)], dbuf, dsem)

  def wait_blk(sbuf, dbuf, ssem, dsem):
    pltpu.make_async_copy(srcp.at[s, pl.ds(0, BPB)], sbuf, ssem).wait()
    pltpu.make_async_copy(dstp.at[s, pl.ds(0, BPB)], dbuf, dsem).wait()

  def gwait(p):
    pltpu.make_async_copy(tbl.at[srcbA.at[0]], rows.at[p], gs[p]).wait()

  def scat(rbuf, didx):
    pltpu.sync_copy(rbuf, acc.at[didx], add=True)
    if dacc is not None:
      pltpu.sync_copy(ones, dacc.at[didx], add=True)

  def half(k, cur_s, cur_d, nxt_s, nxt_d, nssem, ndsem, last_guard):
    for j in range(BPB):
      p = j % 2
      q = 1 - p
      if j < BPB - 1:
        pltpu.async_copy(tbl.at[cur_s.at[j + 1]], rows.at[q], gs[q])
      else:
        @pl.when(last_guard)
        def _():
          wait_blk(nxt_s, nxt_d, nssem, ndsem)
          pltpu.async_copy(tbl.at[nxt_s.at[0]], rows.at[q], gs[q])
      gwait(p)
      scat(rows.at[p], cur_d.at[j])

  def body(k, _):
    always = k >= 0
    half(k, srcbA, dstbA, srcbB, dstbB, sB, dB, always)

    @pl.when(k < NBODY - 1)
    def _():
      load_blk(2 * k + 2, srcbA, dstbA, sA, dA)

    half(k, srcbB, dstbB, srcbA, dstbA, sA, dA, k < NBODY - 1)

    @pl.when(k < NBODY - 1)
    def _():
      load_blk(2 * k + 3, srcbB, dstbB, sB, dB)
    return 0

  lax.fori_loop(0, NBODY, body, 0)
  plsc.subcore_barrier()

  @pl.when(s < NS - 1)
  def _():
    pltpu.sync_copy(acc.at[pl.ds(s * WPT, WPT)], out.at[pl.ds(s * WPT, WPT)])

  @pl.when(s == NS - 1)
  def _():
    base = (NS - 1) * WPT
    pltpu.sync_copy(acc.at[pl.ds(base, WLAST)], out.at[pl.ds(base, WLAST)])

  if dacc is not None:
    pltpu.sync_copy(dacc.at[pl.ds(s * ZPT, ZPT)], degout.at[pl.ds(s * ZPT, ZPT)])


def _make_agg(with_deg):
  out_type = [jax.ShapeDtypeStruct((N, H), jnp.float32),
              jax.ShapeDtypeStruct((N, H), jnp.float32)]
  scratch = [
      pltpu.VMEM((BPB, C), jnp.int32),
      pltpu.VMEM((BPB, C), jnp.int32),
      pltpu.VMEM((BPB, C), jnp.int32),
      pltpu.VMEM((BPB, C), jnp.int32),
      pltpu.VMEM((2, C, H), jnp.float32),
      pltpu.VMEM_SHARED((ACC_R, H), jnp.float32),
  ] + [pltpu.SemaphoreType.DMA] * 6
  if with_deg:
    out_type.append(jax.ShapeDtypeStruct((ACC_R,), jnp.float32))
    scratch += [
        pltpu.VMEM((C,), jnp.float32),
        pltpu.VMEM_SHARED((ACC_R,), jnp.float32),
    ]

  mesh = plsc.VectorSubcoreMesh(core_axis_name="c", subcore_axis_name="s")

  @functools.partial(pl.kernel, mesh=mesh, out_type=out_type,
                     scratch_types=scratch)
  def agg(tA, tB, srcp, dstp, outA, outB, *rest):
    if with_deg:
      (degout, srcA, srcB, dstA, dstB, rows, acc,
       gs0, gs1, isA, idA, isB, idB, ones, dacc) = rest
    else:
      srcA, srcB, dstA, dstB, rows, acc, gs0, gs1, isA, idA, isB, idB = rest
    idx = (srcA, srcB, dstA, dstB)
    sems = (gs0, gs1, isA, idA, isB, idB)
    c = lax.axis_index("c")
    s = lax.axis_index("s")

    @pl.when(c == 0)
    def _():
      _edge_prime(srcp, dstp, tA, idx, rows, sems, s)

    @pl.when(c == 1)
    def _():
      _edge_prime(srcp, dstp, tB, idx, rows, sems, s)

    _zero_vmem_rows(rows.at[1], ZR, H)
    for j in range(ZPT // ZR):
      pltpu.sync_copy(rows.at[1], acc.at[pl.ds(s * ZPT + j * ZR, ZR)])
    if with_deg:
      @pl.when(c == 0)
      def _():
        for k in range(C // 16):
          ones[pl.ds(k * 16, 16)] = jnp.zeros((16,), jnp.float32)
        for j in range(ZPT // C):
          pltpu.sync_copy(ones, dacc.at[pl.ds(s * ZPT + j * C, C)])
        for k in range(C // 16):
          ones[pl.ds(k * 16, 16)] = jnp.ones((16,), jnp.float32)
    plsc.subcore_barrier()

    @pl.when(c == 0)
    def _():
      if with_deg:
        _edge_pass(srcp, dstp, tA, outA, acc, idx, rows, sems, s,
                   dacc=dacc, ones=ones, degout=degout)
      else:
        _edge_pass(srcp, dstp, tA, outA, acc, idx, rows, sems, s)

    @pl.when(c == 1)
    def _():
      _edge_pass(srcp, dstp, tB, outB, acc, idx, rows, sems, s)

  return agg


_agg_deg = _make_agg(True)
_agg = _make_agg(False)


RB = 1000
GRID = N // RB


def _layer1_body(x, aggA, aggB, deg, W1, b1, outA, outB):
  mean = jnp.concatenate([aggA[...], aggB[...]], axis=1) / jnp.maximum(
      deg[...], 1.0)
  h = x[...] + mean
  y = jnp.maximum(
      jax.lax.dot_general(h, W1[...], (((1,), (0,)), ((), ())),
                          preferred_element_type=jnp.float32) + b1[...], 0.0)
  outA[...] = y[:, :H]
  outB[...] = y[:, H:]


def _tc_layer1(x, aggA, aggB, deg, W1, b1):
  return pl.pallas_call(
      _layer1_body,
      grid=(GRID,),
      in_specs=[
          pl.BlockSpec((RB, D), lambda i: (i, 0)),
          pl.BlockSpec((RB, H), lambda i: (i, 0)),
          pl.BlockSpec((RB, H), lambda i: (i, 0)),
          pl.BlockSpec((RB, 1), lambda i: (i, 0)),
          pl.BlockSpec((D, D), lambda i: (0, 0)),
          pl.BlockSpec((1, D), lambda i: (0, 0)),
      ],
      out_specs=[
          pl.BlockSpec((RB, H), lambda i: (i, 0)),
          pl.BlockSpec((RB, H), lambda i: (i, 0)),
      ],
      out_shape=[jax.ShapeDtypeStruct((N, H), jnp.float32),
                 jax.ShapeDtypeStruct((N, H), jnp.float32)],
  )(x, aggA, aggB, deg, W1, b1)


def _layer2_body(hA, hB, aggA, aggB, deg, W2, b2, Wc, bc, out, vmax):
  i = pl.program_id(0)
  d = jnp.maximum(deg[...], 1.0)
  h = (jnp.concatenate([hA[...], hB[...]], axis=1)
       + jnp.concatenate([aggA[...], aggB[...]], axis=1) / d)
  y = jax.lax.dot_general(h, W2[...], (((1,), (0,)), ((), ())),
                          preferred_element_type=jnp.float32) + b2[...]
  m = jnp.max(y, axis=0, keepdims=True)

  @pl.when(i == 0)
  def _():
    vmax[...] = m

  @pl.when(i > 0)
  def _():
    vmax[...] = jnp.maximum(vmax[...], m)

  @pl.when(i == GRID - 1)
  def _():
    out[...] = jax.lax.dot_general(
        vmax[...], Wc[...], (((1,), (0,)), ((), ())),
        preferred_element_type=jnp.float32) + bc[...]


def _tc_layer2(hA, hB, aggA, aggB, deg, W2, b2, Wc, bc):
  return pl.pallas_call(
      _layer2_body,
      grid=(GRID,),
      in_specs=[
          pl.BlockSpec((RB, H), lambda i: (i, 0)),
          pl.BlockSpec((RB, H), lambda i: (i, 0)),
          pl.BlockSpec((RB, H), lambda i: (i, 0)),
          pl.BlockSpec((RB, H), lambda i: (i, 0)),
          pl.BlockSpec((RB, 1), lambda i: (i, 0)),
          pl.BlockSpec((D, D), lambda i: (0, 0)),
          pl.BlockSpec((1, D), lambda i: (0, 0)),
          pl.BlockSpec((D, NCLS), lambda i: (0, 0)),
          pl.BlockSpec((1, NCLS), lambda i: (0, 0)),
      ],
      out_specs=pl.BlockSpec((1, NCLS), lambda i: (0, 0)),
      out_shape=jax.ShapeDtypeStruct((1, NCLS), jnp.float32),
      scratch_shapes=[pltpu.VMEM((1, D), jnp.float32)],
  )(hA, hB, aggA, aggB, deg, W2, b2, Wc, bc)


def kernel(x, edge_index, W1, b1, W2, b2, Wc, bc):
  src = edge_index[0]
  dst = edge_index[1]
  npad = E_PAD - E
  pad_i = jnp.arange(npad, dtype=jnp.int32)
  srcp = jnp.concatenate([src, pad_i % N]).reshape(NS, ITERS, C)
  dstp = jnp.concatenate([dst, N + (pad_i % PADR)]).reshape(NS, ITERS, C)

  xA = x[:, :H]
  xB = x[:, H:]
  aggA, aggB, degw = _agg_deg(xA, xB, srcp, dstp)
  deg = degw[:N].reshape(N, 1)

  h1A, h1B = _tc_layer1(x, aggA, aggB, deg, W1, b1.reshape(1, D))
  a2A, a2B = _agg(h1A, h1B, srcp, dstp)
  return _tc_layer2(h1A, h1B, a2A, a2B, deg, W2, b2.reshape(1, D),
                    Wc, bc.reshape(1, NCLS))


# speedup vs baseline: 9.5838x; 1.0018x over previous
import functools

import jax
import jax.numpy as jnp
from jax import lax
from jax.experimental import pallas as pl
from jax.experimental.pallas import tpu as pltpu
from jax.experimental.pallas import tpu_sc as plsc

N = 10000
E = 160000
D = 256
H = 128
NCLS = 40

NC = 2
NS = 16

C = 128
EP_T = 10240
E_PAD = NS * EP_T
ITERS = EP_T // C
BPB = 8
NBODY = ITERS // (2 * BPB)
PADR = 240
ACC_R = N + PADR
ZPT = ACC_R // NS
ZR = 128
WPT = 624
WLAST = N - (NS - 1) * WPT


def _zero_vmem_rows(ref, nrows, width):
  def body(r, _):
    for k in range(width // 16):
      ref[r, pl.ds(k * 16, 16)] = jnp.zeros((16,), jnp.float32)
    return 0
  lax.fori_loop(0, nrows, body, 0)


def _edge_prime(srcp, dstp, tbl, idx, rows, sems, s):
  srcbA, srcbB, dstbA, dstbB = idx
  gs = sems[0:2]
  sA, dA, sB, dB = sems[2:6]
  pltpu.sync_copy(srcp.at[s, pl.ds(0, BPB)], srcbA)
  pltpu.sync_copy(dstp.at[s, pl.ds(0, BPB)], dstbA)
  pltpu.async_copy(tbl.at[srcbA.at[0]], rows.at[0], gs[0])
  pltpu.async_copy(srcp.at[s, pl.ds(BPB, BPB)], srcbB, sB)
  pltpu.async_copy(dstp.at[s, pl.ds(BPB, BPB)], dstbB, dB)


def _edge_pass(srcp, dstp, tbl, out, acc, idx, rows, sems, s,
               dacc=None, ones=None, degout=None):
  srcbA, srcbB, dstbA, dstbB = idx
  gs = sems[0:2]
  sA, dA, sB, dB = sems[2:6]

  def load_blk(bi, sbuf, dbuf, ssem, dsem):
    pltpu.async_copy(srcp.at[s, pl.ds(bi * BPB, BPB)], sbuf, ssem)
    pltpu.async_copy(dstp.at[s, pl.ds(bi * BPB, BPB)], dbuf, dsem)

  def wait_blk(sbuf, dbuf, ssem, dsem):
    pltpu.make_async_copy(srcp.at[s, pl.ds(0, BPB)], sbuf, ssem).wait()
    pltpu.make_async_copy(dstp.at[s, pl.ds(0, BPB)], dbuf, dsem).wait()

  ss = sems[6:8]

  def gwait(p):
    pltpu.make_async_copy(tbl.at[srcbA.at[0]], rows.at[p], gs[p]).wait()

  def scat_async(rbuf, didx, p):
    pltpu.async_copy(rbuf, acc.at[didx], ss[p], add=True)
    if dacc is not None:
      pltpu.async_copy(ones, dacc.at[didx], ss[p], add=True)

  def scat_sync(rbuf, didx):
    pltpu.sync_copy(rbuf, acc.at[didx], add=True)
    if dacc is not None:
      pltpu.sync_copy(ones, dacc.at[didx], add=True)

  def swait(p):
    pltpu.make_async_copy(tbl.at[srcbA.at[0]], rows.at[p], ss[p]).wait()
    if dacc is not None:
      pltpu.make_async_copy(degout.at[pl.ds(0, C)], ones, ss[p]).wait()

  def half(cur_s, cur_d, nxt_s, nxt_d, nssem, ndsem, last_guard):
    for j in range(BPB):
      p = j % 2
      q = 1 - p
      if j < BPB - 1:
        if j >= 1:
          swait(q)
        pltpu.async_copy(tbl.at[cur_s.at[j + 1]], rows.at[q], gs[q])
      else:
        @pl.when(last_guard)
        def _():
          wait_blk(nxt_s, nxt_d, nssem, ndsem)
          pltpu.async_copy(tbl.at[nxt_s.at[0]], rows.at[q], gs[q])
      gwait(p)
      if j < BPB - 2:
        scat_async(rows.at[p], cur_d.at[j], p)
      else:
        scat_sync(rows.at[p], cur_d.at[j])

  def body(k, _):
    always = k >= 0
    half(srcbA, dstbA, srcbB, dstbB, sB, dB, always)

    @pl.when(k < NBODY - 1)
    def _():
      load_blk(2 * k + 2, srcbA, dstbA, sA, dA)

    half(srcbB, dstbB, srcbA, dstbA, sA, dA, k < NBODY - 1)

    @pl.when(k < NBODY - 1)
    def _():
      load_blk(2 * k + 3, srcbB, dstbB, sB, dB)
    return 0

  lax.fori_loop(0, NBODY, body, 0)
  plsc.subcore_barrier()

  @pl.when(s < NS - 1)
  def _():
    pltpu.sync_copy(acc.at[pl.ds(s * WPT, WPT)], out.at[pl.ds(s * WPT, WPT)])

  @pl.when(s == NS - 1)
  def _():
    base = (NS - 1) * WPT
    pltpu.sync_copy(acc.at[pl.ds(base, WLAST)], out.at[pl.ds(base, WLAST)])

  if dacc is not None:
    pltpu.sync_copy(dacc.at[pl.ds(s * ZPT, ZPT)], degout.at[pl.ds(s * ZPT, ZPT)])


def _make_agg(with_deg):
  out_type = [jax.ShapeDtypeStruct((N, H), jnp.float32),
              jax.ShapeDtypeStruct((N, H), jnp.float32)]
  scratch = [
      pltpu.VMEM((BPB, C), jnp.int32),
      pltpu.VMEM((BPB, C), jnp.int32),
      pltpu.VMEM((BPB, C), jnp.int32),
      pltpu.VMEM((BPB, C), jnp.int32),
      pltpu.VMEM((2, C, H), jnp.float32),
      pltpu.VMEM_SHARED((ACC_R, H), jnp.float32),
  ] + [pltpu.SemaphoreType.DMA] * 8
  if with_deg:
    out_type.append(jax.ShapeDtypeStruct((ACC_R,), jnp.float32))
    scratch += [
        pltpu.VMEM((C,), jnp.float32),
        pltpu.VMEM_SHARED((ACC_R,), jnp.float32),
    ]

  mesh = plsc.VectorSubcoreMesh(core_axis_name="c", subcore_axis_name="s")

  @functools.partial(pl.kernel, mesh=mesh, out_type=out_type,
                     scratch_types=scratch)
  def agg(tA, tB, srcp, dstp, outA, outB, *rest):
    if with_deg:
      (degout, srcA, srcB, dstA, dstB, rows, acc,
       gs0, gs1, isA, idA, isB, idB, ss0, ss1, ones, dacc) = rest
    else:
      (srcA, srcB, dstA, dstB, rows, acc,
       gs0, gs1, isA, idA, isB, idB, ss0, ss1) = rest
    idx = (srcA, srcB, dstA, dstB)
    sems = (gs0, gs1, isA, idA, isB, idB, ss0, ss1)
    c = lax.axis_index("c")
    s = lax.axis_index("s")

    @pl.when(c == 0)
    def _():
      _edge_prime(srcp, dstp, tA, idx, rows, sems, s)

    @pl.when(c == 1)
    def _():
      _edge_prime(srcp, dstp, tB, idx, rows, sems, s)

    _zero_vmem_rows(rows.at[1], ZR, H)
    for j in range(ZPT // ZR):
      pltpu.sync_copy(rows.at[1], acc.at[pl.ds(s * ZPT + j * ZR, ZR)])
    if with_deg:
      @pl.when(c == 1)
      def _():
        for k in range(C // 16):
          ones[pl.ds(k * 16, 16)] = jnp.zeros((16,), jnp.float32)
        for j in range(ZPT // C):
          pltpu.sync_copy(ones, dacc.at[pl.ds(s * ZPT + j * C, C)])
        for k in range(C // 16):
          ones[pl.ds(k * 16, 16)] = jnp.ones((16,), jnp.float32)
    plsc.subcore_barrier()

    @pl.when(c == 0)
    def _():
      _edge_pass(srcp, dstp, tA, outA, acc, idx, rows, sems, s)

    @pl.when(c == 1)
    def _():
      if with_deg:
        _edge_pass(srcp, dstp, tB, outB, acc, idx, rows, sems, s,
                   dacc=dacc, ones=ones, degout=degout)
      else:
        _edge_pass(srcp, dstp, tB, outB, acc, idx, rows, sems, s)

  return agg


_agg_deg = _make_agg(True)
_agg = _make_agg(False)


RB = 1000
GRID = N // RB


def _layer1_body(x, aggA, aggB, deg, W1, b1, outA, outB):
  mean = jnp.concatenate([aggA[...], aggB[...]], axis=1) / jnp.maximum(
      deg[...], 1.0)
  h = x[...] + mean
  y = jnp.maximum(
      jax.lax.dot_general(h, W1[...], (((1,), (0,)), ((), ())),
                          preferred_element_type=jnp.float32) + b1[...], 0.0)
  outA[...] = y[:, :H]
  outB[...] = y[:, H:]


def _tc_layer1(x, aggA, aggB, deg, W1, b1):
  return pl.pallas_call(
      _layer1_body,
      grid=(GRID,),
      in_specs=[
          pl.BlockSpec((RB, D), lambda i: (i, 0)),
          pl.BlockSpec((RB, H), lambda i: (i, 0)),
          pl.BlockSpec((RB, H), lambda i: (i, 0)),
          pl.BlockSpec((RB, 1), lambda i: (i, 0)),
          pl.BlockSpec((D, D), lambda i: (0, 0)),
          pl.BlockSpec((1, D), lambda i: (0, 0)),
      ],
      out_specs=[
          pl.BlockSpec((RB, H), lambda i: (i, 0)),
          pl.BlockSpec((RB, H), lambda i: (i, 0)),
      ],
      out_shape=[jax.ShapeDtypeStruct((N, H), jnp.float32),
                 jax.ShapeDtypeStruct((N, H), jnp.float32)],
  )(x, aggA, aggB, deg, W1, b1)


def _layer2_body(hA, hB, aggA, aggB, deg, W2, b2, Wc, bc, out, vmax):
  i = pl.program_id(0)
  d = jnp.maximum(deg[...], 1.0)
  h = (jnp.concatenate([hA[...], hB[...]], axis=1)
       + jnp.concatenate([aggA[...], aggB[...]], axis=1) / d)
  y = jax.lax.dot_general(h, W2[...], (((1,), (0,)), ((), ())),
                          preferred_element_type=jnp.float32) + b2[...]
  m = jnp.max(y, axis=0, keepdims=True)

  @pl.when(i == 0)
  def _():
    vmax[...] = m

  @pl.when(i > 0)
  def _():
    vmax[...] = jnp.maximum(vmax[...], m)

  @pl.when(i == GRID - 1)
  def _():
    out[...] = jax.lax.dot_general(
        vmax[...], Wc[...], (((1,), (0,)), ((), ())),
        preferred_element_type=jnp.float32) + bc[...]


def _tc_layer2(hA, hB, aggA, aggB, deg, W2, b2, Wc, bc):
  return pl.pallas_call(
      _layer2_body,
      grid=(GRID,),
      in_specs=[
          pl.BlockSpec((RB, H), lambda i: (i, 0)),
          pl.BlockSpec((RB, H), lambda i: (i, 0)),
          pl.BlockSpec((RB, H), lambda i: (i, 0)),
          pl.BlockSpec((RB, H), lambda i: (i, 0)),
          pl.BlockSpec((RB, 1), lambda i: (i, 0)),
          pl.BlockSpec((D, D), lambda i: (0, 0)),
          pl.BlockSpec((1, D), lambda i: (0, 0)),
          pl.BlockSpec((D, NCLS), lambda i: (0, 0)),
          pl.BlockSpec((1, NCLS), lambda i: (0, 0)),
      ],
      out_specs=pl.BlockSpec((1, NCLS), lambda i: (0, 0)),
      out_shape=jax.ShapeDtypeStruct((1, NCLS), jnp.float32),
      scratch_shapes=[pltpu.VMEM((1, D), jnp.float32)],
  )(hA, hB, aggA, aggB, deg, W2, b2, Wc, bc)


def kernel(x, edge_index, W1, b1, W2, b2, Wc, bc):
  src = edge_index[0]
  dst = edge_index[1]
  npad = E_PAD - E
  pad_i = jnp.arange(npad, dtype=jnp.int32)
  srcp = jnp.concatenate([src, pad_i % N]).reshape(NS, ITERS, C)
  dstp = jnp.concatenate([dst, N + (pad_i % PADR)]).reshape(NS, ITERS, C)

  xA = x[:, :H]
  xB = x[:, H:]
  aggA, aggB, degw = _agg_deg(xA, xB, srcp, dstp)
  deg = degw[:N].reshape(N, 1)

  h1A, h1B = _tc_layer1(x, aggA, aggB, deg, W1, b1.reshape(1, D))
  a2A, a2B = _agg(h1A, h1B, srcp, dstp)
  return _tc_layer2(h1A, h1B, a2A, a2B, deg, W2, b2.reshape(1, D),
                    Wc, bc.reshape(1, NCLS))


# speedup vs baseline: 9.7120x; 1.0134x over previous
import functools

import jax
import jax.numpy as jnp
from jax import lax
from jax.experimental import pallas as pl
from jax.experimental.pallas import tpu as pltpu
from jax.experimental.pallas import tpu_sc as plsc

N = 10000
E = 160000
D = 256
H = 128
NCLS = 40

NC = 2
NS = 16

C = 128
EP_T = 10240
E_PAD = NS * EP_T
ITERS = EP_T // C
BPB = 8
NBODY = ITERS // (2 * BPB)
PADR = 240
ACC_R = N + PADR
ZPT = ACC_R // NS
ZR = 128
WPT = 624
WLAST = N - (NS - 1) * WPT


def _zero_vmem_rows(ref, nrows, width):
  def body(r, _):
    for k in range(width // 16):
      ref[r, pl.ds(k * 16, 16)] = jnp.zeros((16,), jnp.float32)
    return 0
  lax.fori_loop(0, nrows, body, 0)


def _edge_prime(srcp, dstp, tbl, idx, rows, sems, s):
  srcbA, srcbB, dstbA, dstbB = idx
  gs = sems[0:2]
  sA, dA, sB, dB = sems[2:6]
  pltpu.sync_copy(srcp.at[s, pl.ds(0, BPB)], srcbA)
  pltpu.sync_copy(dstp.at[s, pl.ds(0, BPB)], dstbA)
  pltpu.async_copy(tbl.at[srcbA.at[0]], rows.at[0], gs[0])
  pltpu.async_copy(srcp.at[s, pl.ds(BPB, BPB)], srcbB, sB)
  pltpu.async_copy(dstp.at[s, pl.ds(BPB, BPB)], dstbB, dB)


def _edge_pass(srcp, dstp, tbl, out, acc, idx, rows, sems, s,
               dacc=None, ones=None, degout=None):
  srcbA, srcbB, dstbA, dstbB = idx
  gs = sems[0:2]
  sA, dA, sB, dB = sems[2:6]

  def load_blk(bi, sbuf, dbuf, ssem, dsem):
    pltpu.async_copy(srcp.at[s, pl.ds(bi * BPB, BPB)], sbuf, ssem)
    pltpu.async_copy(dstp.at[s, pl.ds(bi * BPB, BPB)], dbuf, dsem)

  def wait_blk(sbuf, dbuf, ssem, dsem):
    pltpu.make_async_copy(srcp.at[s, pl.ds(0, BPB)], sbuf, ssem).wait()
    pltpu.make_async_copy(dstp.at[s, pl.ds(0, BPB)], dbuf, dsem).wait()

  ss = sems[6:8]

  def gwait(p):
    pltpu.make_async_copy(tbl.at[srcbA.at[0]], rows.at[p], gs[p]).wait()

  def scat_async(rbuf, didx, p):
    pltpu.async_copy(rbuf, acc.at[didx], ss[p], add=True)
    if dacc is not None:
      pltpu.async_copy(ones, dacc.at[didx], ss[p], add=True)

  def scat_sync(rbuf, didx):
    pltpu.sync_copy(rbuf, acc.at[didx], add=True)
    if dacc is not None:
      pltpu.sync_copy(ones, dacc.at[didx], add=True)

  def swait(p):
    pltpu.make_async_copy(tbl.at[srcbA.at[0]], rows.at[p], ss[p]).wait()
    if dacc is not None:
      pltpu.make_async_copy(degout.at[pl.ds(0, C)], ones, ss[p]).wait()

  def half(cur_s, cur_d, nxt_s, nxt_d, nssem, ndsem, last_guard):
    for j in range(BPB):
      p = j % 2
      q = 1 - p
      if j < BPB - 1:
        if j >= 1:
          swait(q)
        pltpu.async_copy(tbl.at[cur_s.at[j + 1]], rows.at[q], gs[q])
      else:
        @pl.when(last_guard)
        def _():
          wait_blk(nxt_s, nxt_d, nssem, ndsem)
          pltpu.async_copy(tbl.at[nxt_s.at[0]], rows.at[q], gs[q])
      gwait(p)
      if j < BPB - 2:
        scat_async(rows.at[p], cur_d.at[j], p)
      else:
        scat_sync(rows.at[p], cur_d.at[j])

  def body(k, _):
    always = k >= 0
    half(srcbA, dstbA, srcbB, dstbB, sB, dB, always)

    @pl.when(k < NBODY - 1)
    def _():
      load_blk(2 * k + 2, srcbA, dstbA, sA, dA)

    half(srcbB, dstbB, srcbA, dstbA, sA, dA, k < NBODY - 1)

    @pl.when(k < NBODY - 1)
    def _():
      load_blk(2 * k + 3, srcbB, dstbB, sB, dB)
    return 0

  lax.fori_loop(0, NBODY, body, 0)
  plsc.subcore_barrier()

  @pl.when(s < NS - 1)
  def _():
    pltpu.sync_copy(acc.at[pl.ds(s * WPT, WPT)], out.at[pl.ds(s * WPT, WPT)])

  @pl.when(s == NS - 1)
  def _():
    base = (NS - 1) * WPT
    pltpu.sync_copy(acc.at[pl.ds(base, WLAST)], out.at[pl.ds(base, WLAST)])

  if dacc is not None:
    pltpu.sync_copy(dacc.at[pl.ds(s * ZPT, ZPT)], degout.at[pl.ds(s * ZPT, ZPT)])


def _make_agg(with_deg):
  out_type = [jax.ShapeDtypeStruct((N, H), jnp.float32),
              jax.ShapeDtypeStruct((N, H), jnp.float32)]
  scratch = [
      pltpu.VMEM((BPB, C), jnp.int32),
      pltpu.VMEM((BPB, C), jnp.int32),
      pltpu.VMEM((BPB, C), jnp.int32),
      pltpu.VMEM((BPB, C), jnp.int32),
      pltpu.VMEM((2, C, H), jnp.float32),
      pltpu.VMEM_SHARED((ACC_R, H), jnp.float32),
  ] + [pltpu.SemaphoreType.DMA] * 8
  if with_deg:
    out_type.append(jax.ShapeDtypeStruct((ACC_R,), jnp.float32))
    scratch += [
        pltpu.VMEM((C,), jnp.float32),
        pltpu.VMEM_SHARED((ACC_R,), jnp.float32),
    ]

  mesh = plsc.VectorSubcoreMesh(core_axis_name="c", subcore_axis_name="s")

  @functools.partial(pl.kernel, mesh=mesh, out_type=out_type,
                     scratch_types=scratch)
  def agg(tA, tB, srcp, dstp, outA, outB, *rest):
    if with_deg:
      (degout, srcA, srcB, dstA, dstB, rows, acc,
       gs0, gs1, isA, idA, isB, idB, ss0, ss1, ones, dacc) = rest
    else:
      (srcA, srcB, dstA, dstB, rows, acc,
       gs0, gs1, isA, idA, isB, idB, ss0, ss1) = rest
    idx = (srcA, srcB, dstA, dstB)
    sems = (gs0, gs1, isA, idA, isB, idB, ss0, ss1)
    c = lax.axis_index("c")
    s = lax.axis_index("s")

    @pl.when(c == 0)
    def _():
      _edge_prime(srcp, dstp, tA, idx, rows, sems, s)

    @pl.when(c == 1)
    def _():
      _edge_prime(srcp, dstp, tB, idx, rows, sems, s)

    _zero_vmem_rows(rows.at[1], ZR, H)
    for j in range(ZPT // ZR):
      pltpu.sync_copy(rows.at[1], acc.at[pl.ds(s * ZPT + j * ZR, ZR)])
    if with_deg:
      @pl.when(c == 1)
      def _():
        for k in range(C // 16):
          ones[pl.ds(k * 16, 16)] = jnp.zeros((16,), jnp.float32)
        for j in range(ZPT // C):
          pltpu.sync_copy(ones, dacc.at[pl.ds(s * ZPT + j * C, C)])
        for k in range(C // 16):
          ones[pl.ds(k * 16, 16)] = jnp.ones((16,), jnp.float32)
    plsc.subcore_barrier()

    @pl.when(c == 0)
    def _():
      _edge_pass(srcp, dstp, tA, outA, acc, idx, rows, sems, s)

    @pl.when(c == 1)
    def _():
      if with_deg:
        _edge_pass(srcp, dstp, tB, outB, acc, idx, rows, sems, s,
                   dacc=dacc, ones=ones, degout=degout)
      else:
        _edge_pass(srcp, dstp, tB, outB, acc, idx, rows, sems, s)

  return agg


_agg_deg = _make_agg(True)
_agg = _make_agg(False)


RB = 2000
GRID = N // RB


def _layer1_body(x, aggA, aggB, deg, W1, b1, outA, outB):
  mean = jnp.concatenate([aggA[...], aggB[...]], axis=1) / jnp.maximum(
      deg[...], 1.0)
  h = x[...] + mean
  y = jnp.maximum(
      jax.lax.dot_general(h, W1[...], (((1,), (0,)), ((), ())),
                          preferred_element_type=jnp.float32) + b1[...], 0.0)
  outA[...] = y[:, :H]
  outB[...] = y[:, H:]


def _tc_layer1(x, aggA, aggB, deg, W1, b1):
  return pl.pallas_call(
      _layer1_body,
      grid=(GRID,),
      in_specs=[
          pl.BlockSpec((RB, D), lambda i: (i, 0)),
          pl.BlockSpec((RB, H), lambda i: (i, 0)),
          pl.BlockSpec((RB, H), lambda i: (i, 0)),
          pl.BlockSpec((RB, 1), lambda i: (i, 0)),
          pl.BlockSpec((D, D), lambda i: (0, 0)),
          pl.BlockSpec((1, D), lambda i: (0, 0)),
      ],
      out_specs=[
          pl.BlockSpec((RB, H), lambda i: (i, 0)),
          pl.BlockSpec((RB, H), lambda i: (i, 0)),
      ],
      out_shape=[jax.ShapeDtypeStruct((N, H), jnp.float32),
                 jax.ShapeDtypeStruct((N, H), jnp.float32)],
  )(x, aggA, aggB, deg, W1, b1)


def _layer2_body(hA, hB, aggA, aggB, deg, W2, b2, Wc, bc, out, vmax):
  i = pl.program_id(0)
  d = jnp.maximum(deg[...], 1.0)
  h = (jnp.concatenate([hA[...], hB[...]], axis=1)
       + jnp.concatenate([aggA[...], aggB[...]], axis=1) / d)
  y = jax.lax.dot_general(h, W2[...], (((1,), (0,)), ((), ())),
                          preferred_element_type=jnp.float32) + b2[...]
  m = jnp.max(y, axis=0, keepdims=True)

  @pl.when(i == 0)
  def _():
    vmax[...] = m

  @pl.when(i > 0)
  def _():
    vmax[...] = jnp.maximum(vmax[...], m)

  @pl.when(i == GRID - 1)
  def _():
    out[...] = jax.lax.dot_general(
        vmax[...], Wc[...], (((1,), (0,)), ((), ())),
        preferred_element_type=jnp.float32) + bc[...]


def _tc_layer2(hA, hB, aggA, aggB, deg, W2, b2, Wc, bc):
  return pl.pallas_call(
      _layer2_body,
      grid=(GRID,),
      in_specs=[
          pl.BlockSpec((RB, H), lambda i: (i, 0)),
          pl.BlockSpec((RB, H), lambda i: (i, 0)),
          pl.BlockSpec((RB, H), lambda i: (i, 0)),
          pl.BlockSpec((RB, H), lambda i: (i, 0)),
          pl.BlockSpec((RB, 1), lambda i: (i, 0)),
          pl.BlockSpec((D, D), lambda i: (0, 0)),
          pl.BlockSpec((1, D), lambda i: (0, 0)),
          pl.BlockSpec((D, NCLS), lambda i: (0, 0)),
          pl.BlockSpec((1, NCLS), lambda i: (0, 0)),
      ],
      out_specs=pl.BlockSpec((1, NCLS), lambda i: (0, 0)),
      out_shape=jax.ShapeDtypeStruct((1, NCLS), jnp.float32),
      scratch_shapes=[pltpu.VMEM((1, D), jnp.float32)],
  )(hA, hB, aggA, aggB, deg, W2, b2, Wc, bc)


def kernel(x, edge_index, W1, b1, W2, b2, Wc, bc):
  src = edge_index[0]
  dst = edge_index[1]
  npad = E_PAD - E
  pad_i = jnp.arange(npad, dtype=jnp.int32)
  srcp = jnp.concatenate([src, pad_i % N]).reshape(NS, ITERS, C)
  dstp = jnp.concatenate([dst, N + (pad_i % PADR)]).reshape(NS, ITERS, C)

  xA = x[:, :H]
  xB = x[:, H:]
  aggA, aggB, degw = _agg_deg(xA, xB, srcp, dstp)
  deg = degw[:N].reshape(N, 1)

  h1A, h1B = _tc_layer1(x, aggA, aggB, deg, W1, b1.reshape(1, D))
  a2A, a2B = _agg(h1A, h1B, srcp, dstp)
  return _tc_layer2(h1A, h1B, a2A, a2B, deg, W2, b2.reshape(1, D),
                    Wc, bc.reshape(1, NCLS))
